# baseline (speedup 1.0000x reference)
import math

import jax
import jax.numpy as jnp
from jax.experimental import pallas as pl


_SCALE = math.sqrt(1024.0)
_BLK_S = 2048


def _pe_kernel(x_ref, pw_ref, o_ref):
    o_ref[...] = x_ref[...] * _SCALE + pw_ref[...]


def kernel(x, pos_weight):
    batch, seq_len, d_model = x.shape
    n_s = seq_len // _BLK_S
    grid = (n_s, batch)
    return pl.pallas_call(
        _pe_kernel,
        grid=grid,
        in_specs=[
            pl.BlockSpec((1, _BLK_S, d_model), lambda j, b: (b, j, 0)),
            pl.BlockSpec((_BLK_S, d_model), lambda j, b: (j, 0)),
        ],
        out_specs=pl.BlockSpec((1, _BLK_S, d_model), lambda j, b: (b, j, 0)),
        out_shape=jax.ShapeDtypeStruct(x.shape, x.dtype),
    )(x, pos_weight)
